# Initial kernel scaffold; baseline (speedup 1.0000x reference)
import jax
import jax.numpy as jnp
from jax.experimental import pallas as pl


def kernel(relative_position_bias_table, relative_index):
    raise NotImplementedError("write your pallas kernel here")


# speedup vs baseline: 3.4232x; 3.4232x over previous
import jax
import jax.numpy as jnp
from jax import lax
from jax.experimental import pallas as pl
from jax.experimental.pallas import tpu as pltpu
from jax.experimental.pallas import tpu_sc as plsc

_H = 32
_T = 2209
_N = 576 * 576
_NC, _NS, _L = 2, 16, 16
_NW = _NC * _NS
_NPW = _N // _NW
_CHUNKS = _NPW // _L


def _body(tab_hbm, idx_hbm, out_hbm, tab_v, idx_v, val_v):
    w = lax.axis_index("s") * _NC + lax.axis_index("c")
    base = w * _NPW
    pltpu.sync_copy(tab_hbm, tab_v)
    pltpu.sync_copy(idx_hbm.at[pl.ds(base, _NPW)], idx_v)

    def _scale(c, carry):
        s = pl.ds(c * _L, _L)
        idx_v[s] = idx_v[s] * _H
        return carry

    lax.fori_loop(0, _CHUNKS, _scale, 0)

    def _per_head(h, carry):
        hv = jnp.full((_L,), h, jnp.int32)

        def _per_chunk(c, inner):
            s = pl.ds(c * _L, _L)
            val_v[s] = plsc.load_gather(tab_v, [idx_v[s] + hv])
            return inner

        lax.fori_loop(0, _CHUNKS, _per_chunk, 0)
        pltpu.sync_copy(val_v, out_hbm.at[h, pl.ds(base, _NPW)])
        return carry

    lax.fori_loop(0, _H, _per_head, 0)


def kernel(relative_position_bias_table, relative_index):
    tab_flat = relative_position_bias_table.reshape(-1)
    idx_flat = relative_index.reshape(-1)
    mesh = plsc.VectorSubcoreMesh(core_axis_name="c", subcore_axis_name="s")
    out = pl.kernel(
        _body,
        out_type=jax.ShapeDtypeStruct((_H, _N), jnp.float32),
        mesh=mesh,
        scratch_types=[
            pltpu.VMEM((_T * _H,), jnp.float32),
            pltpu.VMEM((_NPW,), jnp.int32),
            pltpu.VMEM((_NPW,), jnp.float32),
        ],
        compiler_params=pltpu.CompilerParams(needs_layout_passes=False),
    )(tab_flat, idx_flat)
    return out.reshape(_H, 576, 576)


# speedup vs baseline: 3.5212x; 1.0286x over previous
import jax
import jax.numpy as jnp
from jax import lax
from jax.experimental import pallas as pl
from jax.experimental.pallas import tpu as pltpu
from jax.experimental.pallas import tpu_sc as plsc

_H = 32
_T = 2209
_N = 576 * 576
_NC, _NS, _L = 2, 16, 16
_NW = _NC * _NS
_NPW = _N // _NW
_CHUNKS = _NPW // _L
_SUB = 384
_CPS = _SUB // _L
_NBANDS = _NPW // _SUB


def _body(tab_hbm, idx_hbm, out_hbm, tab_v, idx_v, ob0, ob1, sem0, sem1):
    w = lax.axis_index("s") * _NC + lax.axis_index("c")
    base = w * _NPW
    pltpu.sync_copy(tab_hbm, tab_v)
    pltpu.sync_copy(idx_hbm.at[pl.ds(base, _NPW)], idx_v)

    def _scale(c, carry):
        s = pl.ds(c * _L, _L)
        idx_v[s] = idx_v[s] * _H
        return carry

    lax.fori_loop(0, _CHUNKS, _scale, 0)

    def _fill(r, ob):
        def _chunk(c, carry):
            iv = idx_v[pl.ds(r * _SUB + c * _L, _L)]
            s = pl.ds(c * _L, _L)
            for h in range(_H):
                ob[h, s] = plsc.load_gather(tab_v, [iv + h])
            return carry

        lax.fori_loop(0, _CPS, _chunk, 0)

    def _dst(r):
        return out_hbm.at[:, pl.ds(base + r * _SUB, _SUB)]

    def _outer(r2, carry):
        @pl.when(r2 != 0)
        def _():
            pltpu.make_async_copy(ob0, _dst(0), sem0).wait()

        _fill(2 * r2, ob0)
        pltpu.async_copy(ob0, _dst(2 * r2), sem0)

        @pl.when(r2 != 0)
        def _():
            pltpu.make_async_copy(ob1, _dst(0), sem1).wait()

        _fill(2 * r2 + 1, ob1)
        pltpu.async_copy(ob1, _dst(2 * r2 + 1), sem1)
        return carry

    lax.fori_loop(0, _NBANDS // 2, _outer, 0)
    pltpu.make_async_copy(ob0, _dst(0), sem0).wait()
    _fill(_NBANDS - 1, ob0)
    pltpu.async_copy(ob0, _dst(_NBANDS - 1), sem0)
    pltpu.make_async_copy(ob0, _dst(0), sem0).wait()
    pltpu.make_async_copy(ob1, _dst(0), sem1).wait()


def kernel(relative_position_bias_table, relative_index):
    tab_flat = relative_position_bias_table.reshape(-1)
    idx_flat = relative_index.reshape(-1)
    mesh = plsc.VectorSubcoreMesh(core_axis_name="c", subcore_axis_name="s")
    out = pl.kernel(
        _body,
        out_type=jax.ShapeDtypeStruct((_H, _N), jnp.float32),
        mesh=mesh,
        scratch_types=[
            pltpu.VMEM((_T * _H,), jnp.float32),
            pltpu.VMEM((_NPW,), jnp.int32),
            pltpu.VMEM((_H, _SUB), jnp.float32),
            pltpu.VMEM((_H, _SUB), jnp.float32),
            pltpu.SemaphoreType.DMA,
            pltpu.SemaphoreType.DMA,
        ],
        compiler_params=pltpu.CompilerParams(needs_layout_passes=False),
    )(tab_flat, idx_flat)
    return out.reshape(_H, 576, 576)


# speedup vs baseline: 6.9139x; 1.9635x over previous
import jax
import jax.numpy as jnp
from jax import lax
from jax.experimental import pallas as pl
from jax.experimental.pallas import tpu as pltpu
from jax.experimental.pallas import tpu_sc as plsc

_H = 32
_T = 2209
_N = 576 * 576
_NC, _NS, _L = 2, 16, 16
_NW = _NC * _NS
_NPW = _N // _NW
_CHUNKS = _NPW // _L
_SUB = 384
_CPS = _SUB // _L
_NBANDS = _NPW // _SUB


def _body(tab_hbm, idx_hbm, out_hbm, tab_v, idx_v, ob0, ob1, sem0, sem1):
    w = lax.axis_index("s") * _NC + lax.axis_index("c")
    base = w * _NPW
    pltpu.sync_copy(tab_hbm, tab_v)
    pltpu.sync_copy(idx_hbm.at[pl.ds(base, _NPW)], idx_v)

    def _fill(r, ob):
        def _chunk(c, carry):
            iv = idx_v[pl.ds(r * _SUB + c * _L, _L)]
            s = pl.ds(c * _L, _L)
            for h in range(_H):
                ob[h, s] = plsc.load_gather(tab_v, [iv + (h * _T)])
            return carry

        lax.fori_loop(0, _CPS, _chunk, 0)

    def _dst(r):
        return out_hbm.at[:, pl.ds(base + r * _SUB, _SUB)]

    def _outer(r2, carry):
        @pl.when(r2 != 0)
        def _():
            pltpu.make_async_copy(ob0, _dst(0), sem0).wait()

        _fill(2 * r2, ob0)
        pltpu.async_copy(ob0, _dst(2 * r2), sem0)

        @pl.when(r2 != 0)
        def _():
            pltpu.make_async_copy(ob1, _dst(0), sem1).wait()

        _fill(2 * r2 + 1, ob1)
        pltpu.async_copy(ob1, _dst(2 * r2 + 1), sem1)
        return carry

    lax.fori_loop(0, _NBANDS // 2, _outer, 0)
    pltpu.make_async_copy(ob0, _dst(0), sem0).wait()
    _fill(_NBANDS - 1, ob0)
    pltpu.async_copy(ob0, _dst(_NBANDS - 1), sem0)
    pltpu.make_async_copy(ob0, _dst(0), sem0).wait()
    pltpu.make_async_copy(ob1, _dst(0), sem1).wait()


def kernel(relative_position_bias_table, relative_index):
    tab_t = relative_position_bias_table.T.reshape(-1)
    idx_flat = relative_index.reshape(-1)
    mesh = plsc.VectorSubcoreMesh(core_axis_name="c", subcore_axis_name="s")
    out = pl.kernel(
        _body,
        out_type=jax.ShapeDtypeStruct((_H, _N), jnp.float32),
        mesh=mesh,
        scratch_types=[
            pltpu.VMEM((_H * _T,), jnp.float32),
            pltpu.VMEM((_NPW,), jnp.int32),
            pltpu.VMEM((_H, _SUB), jnp.float32),
            pltpu.VMEM((_H, _SUB), jnp.float32),
            pltpu.SemaphoreType.DMA,
            pltpu.SemaphoreType.DMA,
        ],
        compiler_params=pltpu.CompilerParams(needs_layout_passes=False),
    )(tab_t, idx_flat)
    return out.reshape(_H, 576, 576)


# speedup vs baseline: 10.7136x; 1.5496x over previous
import jax
import jax.numpy as jnp
from jax import lax
from jax.experimental import pallas as pl
from jax.experimental.pallas import tpu as pltpu
from jax.experimental.pallas import tpu_sc as plsc

_H = 32
_T = 2209
_N = 576 * 576
_NC, _NS, _L = 2, 16, 16
_NW = _NC * _NS
_NPW = _N // _NW
_CHUNKS = _NPW // _L
_SUB = 384
_CPS = _SUB // _L
_NBANDS = _NPW // _SUB


def _body(tab_hbm, idx_hbm, out_hbm, tab_v, idx_v, ob0, ob1, sem0, sem1):
    w = lax.axis_index("s") * _NC + lax.axis_index("c")
    base = w * _NPW
    pltpu.sync_copy(tab_hbm, tab_v)
    pltpu.sync_copy(idx_hbm.at[pl.ds(base, _NPW)], idx_v)

    def _fill(r, ob):
        @plsc.parallel_loop(0, _CPS, 1, unroll=2)
        def _chunk(c):
            iv = idx_v[pl.ds(r * _SUB + c * _L, _L)]
            s = pl.ds(c * _L, _L)
            for h in range(_H):
                ob[h, s] = plsc.load_gather(tab_v, [iv + (h * _T)])

    def _dst(r):
        return out_hbm.at[:, pl.ds(base + r * _SUB, _SUB)]

    def _outer(r2, carry):
        @pl.when(r2 != 0)
        def _():
            pltpu.make_async_copy(ob0, _dst(0), sem0).wait()

        _fill(2 * r2, ob0)
        pltpu.async_copy(ob0, _dst(2 * r2), sem0)

        @pl.when(r2 != 0)
        def _():
            pltpu.make_async_copy(ob1, _dst(0), sem1).wait()

        _fill(2 * r2 + 1, ob1)
        pltpu.async_copy(ob1, _dst(2 * r2 + 1), sem1)
        return carry

    lax.fori_loop(0, _NBANDS // 2, _outer, 0)
    pltpu.make_async_copy(ob0, _dst(0), sem0).wait()
    _fill(_NBANDS - 1, ob0)
    pltpu.async_copy(ob0, _dst(_NBANDS - 1), sem0)
    pltpu.make_async_copy(ob0, _dst(0), sem0).wait()
    pltpu.make_async_copy(ob1, _dst(0), sem1).wait()


def kernel(relative_position_bias_table, relative_index):
    tab_t = relative_position_bias_table.T.reshape(-1)
    idx_flat = relative_index.reshape(-1)
    mesh = plsc.VectorSubcoreMesh(core_axis_name="c", subcore_axis_name="s")
    out = pl.kernel(
        _body,
        out_type=jax.ShapeDtypeStruct((_H, _N), jnp.float32),
        mesh=mesh,
        scratch_types=[
            pltpu.VMEM((_H * _T,), jnp.float32),
            pltpu.VMEM((_NPW,), jnp.int32),
            pltpu.VMEM((_H, _SUB), jnp.float32),
            pltpu.VMEM((_H, _SUB), jnp.float32),
            pltpu.SemaphoreType.DMA,
            pltpu.SemaphoreType.DMA,
        ],
        compiler_params=pltpu.CompilerParams(needs_layout_passes=False),
    )(tab_t, idx_flat)
    return out.reshape(_H, 576, 576)
